# baseline (speedup 1.0000x reference)
import jax
import jax.numpy as jnp
from jax.experimental import pallas as pl
from jax.experimental.pallas import tpu as pltpu

_BASE = 2
_MAX_K = 10
_NCLS = _MAX_K + 1


def _onehot_body(x_ref, o_ref, bucket_scr):
    c = pl.program_id(1)

    @pl.when(c == 0)
    def _compute_bucket():
        ts = x_ref[...]
        shifted = jnp.concatenate([ts[:1], ts[:-1]], axis=0)
        itvl = ts - shifted
        lg = jnp.log(itvl) / jnp.log(jnp.float32(_BASE))
        lg = jnp.where(jnp.isnan(lg), jnp.zeros_like(lg), lg)
        bucket = jnp.clip(jnp.floor(lg), 0.0, float(_MAX_K)).astype(jnp.int32)
        bucket_scr[...] = bucket.reshape(bucket.shape[0], -1)

    o_ref[0] = (bucket_scr[...] == c).astype(jnp.float32)


@jax.jit
def kernel(timestamps):
    B, T, _ = timestamps.shape
    xin = timestamps.transpose(1, 2, 0).reshape(T, B // 128, 128)
    lb = 1024
    out = pl.pallas_call(
        _onehot_body,
        grid=(B // lb, _NCLS),
        in_specs=[pl.BlockSpec((T, lb // 128, 128), lambda i, c: (0, i, 0))],
        out_specs=pl.BlockSpec((1, T, lb), lambda i, c: (c, 0, i)),
        out_shape=jax.ShapeDtypeStruct((_NCLS, T, B), jnp.float32),
        scratch_shapes=[pltpu.VMEM((T, lb), jnp.int32)],
    )(xin)
    return jnp.transpose(out, (2, 1, 0))


# speedup vs baseline: 2.2108x; 2.2108x over previous
import jax
import jax.numpy as jnp
from jax.experimental import pallas as pl

_BASE = 2
_MAX_K = 10
_NCLS = _MAX_K + 1


def _onehot_body(x_ref, o_ref):
    ts = x_ref[...]
    shifted = jnp.concatenate([ts[:1], ts[:-1]], axis=0)
    itvl = ts - shifted
    lg = jnp.log(itvl) / jnp.log(jnp.float32(_BASE))
    lg = jnp.where(jnp.isnan(lg), jnp.zeros_like(lg), lg)
    bucket = jnp.clip(jnp.floor(lg), 0.0, float(_MAX_K)).astype(jnp.int32)
    bucket_l = bucket.reshape(bucket.shape[0], -1)
    for c in range(_NCLS):
        o_ref[c] = (bucket_l == c).astype(jnp.float32)


@jax.jit
def kernel(timestamps):
    B, T, _ = timestamps.shape
    xin = timestamps.transpose(1, 2, 0).reshape(T, B // 128, 128)
    lb = 1024
    out = pl.pallas_call(
        _onehot_body,
        grid=(B // lb,),
        in_specs=[pl.BlockSpec((T, lb // 128, 128), lambda i: (0, i, 0))],
        out_specs=pl.BlockSpec((_NCLS, T, lb), lambda i: (0, 0, i)),
        out_shape=jax.ShapeDtypeStruct((_NCLS, T, B), jnp.float32),
    )(xin)
    return jnp.transpose(out, (2, 1, 0))
